# baseline (speedup 1.0000x reference)
import jax
import jax.numpy as jnp
from jax.experimental import pallas as pl

G = 16
N2 = 32
T = G * G
C = 3
N2K = C * N2 * N2


def _patch_kernel(x_ref, m_ref, repl_ref, out_ref):
    nb = x_ref.shape[2] // N2
    x = x_ref[0]
    y = x.reshape(C, nb, N2, G, N2).transpose(1, 3, 0, 2, 4).reshape(nb * G, N2K)
    m = m_ref[0, 0, 0, :]
    repl = repl_ref[0]
    out_ref[0] = jnp.where(m[:, None] > 0.5, repl[None, :], y)


def kernel(X):
    b = X.shape[0]
    k1, k2 = jax.random.split(jax.random.key(1))
    idx = jax.random.bernoulli(k1, 1.0 / T, (b * T,))
    repl = jnp.tanh(jax.random.normal(k2, (N2K,), dtype=jnp.float32))

    NB = 16
    m4 = idx.reshape(b, G // NB, 1, NB * G).astype(jnp.float32)
    repl2 = repl.reshape(1, N2K)

    out = pl.pallas_call(
        _patch_kernel,
        grid=(b, G // NB),
        in_specs=[
            pl.BlockSpec((1, C, NB * N2, G * N2), lambda i, j: (i, 0, j, 0)),
            pl.BlockSpec((1, 1, 1, NB * G), lambda i, j: (i, j, 0, 0)),
            pl.BlockSpec((1, N2K), lambda i, j: (0, 0)),
        ],
        out_specs=pl.BlockSpec((1, NB * G, N2K), lambda i, j: (i, j, 0)),
        out_shape=jax.ShapeDtypeStruct((b, T, N2K), jnp.float32),
    )(X, m4, repl2)

    return out, idx
